# baseline (speedup 1.0000x reference)
import jax
import jax.numpy as jnp
from jax import lax
from jax.experimental import pallas as pl
from jax.experimental.pallas import tpu as pltpu
from jax.experimental.pallas import tpu_sc as plsc

NE = 100000
NR = 500
DIM = 128
B = 4096
L = 50

NC = 2
NS = 16
LANES = 16
NW = NC * NS

NPAIR = B * L
PER_W = NPAIR // NW
CHUNK = 64
SUP = 1280
NSUP = PER_W // SUP
CPS = SUP // CHUNK
NBUF = 2


def _body(u_hbm, v_hbm, r_hbm, t_hbm, q_hbm, bs_hbm, bo_hbm,
          EV_hbm, out_hbm,
          uix, vix, rix, tv, outv, accbuf, rows, qr, bias, sems):
    wid = lax.axis_index("s") * NC + lax.axis_index("c")
    base = wid * PER_W
    lanes = lax.iota(jnp.int32, LANES)

    def copies(g, b):
        csl = pl.ds(g * CHUNK, CHUNK)
        EVu, EVv = rows[b].at[0], rows[b].at[1]
        return [
            pltpu.make_async_copy(EV_hbm.at[uix.at[csl]], EVu, sems.at[b]),
            pltpu.make_async_copy(EV_hbm.at[vix.at[csl]], EVv, sems.at[b]),
            pltpu.make_async_copy(q_hbm.at[rix.at[csl]], qr[b], sems.at[b]),
            pltpu.make_async_copy(bs_hbm.at[uix.at[csl]], bias[b].at[0], sems.at[b]),
            pltpu.make_async_copy(bo_hbm.at[vix.at[csl]], bias[b].at[1], sems.at[b]),
        ]

    def fire(g, b):
        for cp in copies(g, b):
            cp.start()

    def drain(g, b):
        for cp in copies(g, b):
            cp.wait()

    def compute(g, b):
        EVu, EVv = rows[b].at[0], rows[b].at[1]

        def group_body(gi, carry2):
            gsl = pl.ds(gi * LANES, LANES)

            def unp(ref, i, w):
                return plsc.unpack(plsc.bitcast(ref[i, pl.ds(w, LANES)],
                                                jnp.bfloat16),
                                   format=plsc.PackFormat.INTERLEAVED)

            @plsc.parallel_loop(0, LANES, unroll=4)
            def pair_iter(k):
                i = gi * LANES + k
                aa = jnp.zeros((LANES,), jnp.float32)
                ab = jnp.zeros((LANES,), jnp.float32)
                bb = jnp.zeros((LANES,), jnp.float32)
                for j4 in range(DIM // 32):
                    w = LANES * j4
                    eus = unp(EVu, i, w)
                    vus = unp(EVu, i, DIM // 2 + w)
                    evs = unp(EVv, i, w)
                    vvs = unp(EVv, i, DIM // 2 + w)
                    pgs = unp(qr[b], i, w)
                    pps = unp(qr[b], i, DIM // 2 + w)
                    for h in range(2):
                        pg = pgs[h]
                        a = eus[h] * pg - evs[h] - pps[h]
                        bq = vus[h] * pg - vvs[h]
                        aa = aa + a * a
                        ab = ab + a * bq
                        bb = bb + bq * bq
                accbuf[k, pl.ds(0, LANES)] = aa
                accbuf[k + LANES, pl.ds(0, LANES)] = ab
                accbuf[k + 2 * LANES, pl.ds(0, LANES)] = bb

            ta = jnp.zeros((LANES,), jnp.float32)
            tc = jnp.zeros((LANES,), jnp.float32)
            tb = jnp.zeros((LANES,), jnp.float32)
            for c in range(LANES):
                cc = jnp.full((LANES,), c, jnp.int32)
                ta = ta + plsc.load_gather(accbuf, [lanes, cc])
                tc = tc + plsc.load_gather(accbuf, [lanes + LANES, cc])
                tb = tb + plsc.load_gather(accbuf, [lanes + 2 * LANES, cc])
            osl = pl.ds(g * CHUNK + gi * LANES, LANES)
            tg = tv[osl]
            totals = ta + (tc + tc) * tg + tb * tg * tg
            outv[osl] = bias[b][0, gsl] + bias[b][1, gsl] - totals
            return carry2

        lax.fori_loop(0, CHUNK // LANES, group_body, 0, unroll=False)

    def super_body(s, carry):
        ssl = pl.ds(base + s * SUP, SUP)
        pltpu.sync_copy(u_hbm.at[ssl], uix)
        pltpu.sync_copy(v_hbm.at[ssl], vix)
        pltpu.sync_copy(r_hbm.at[ssl], rix)
        pltpu.sync_copy(t_hbm.at[ssl], tv)

        for b in range(NBUF):
            fire(b, b)

        def duo_body(g, carry2):
            for b in range(NBUF):
                gg = g + b
                drain(gg, b)
                compute(gg, b)

                @pl.when(gg + NBUF < CPS)
                def _():
                    fire(gg + NBUF, b)
            return carry2

        lax.fori_loop(0, CPS // NBUF, lambda i, c: duo_body(i * NBUF, c), 0,
                      unroll=False)
        pltpu.sync_copy(outv, out_hbm.at[ssl])
        return carry

    lax.fori_loop(0, NSUP, super_body, 0, unroll=False)


@jax.jit
def _run(u, v, r, t, q, bs, bo, EV):
    mesh = plsc.VectorSubcoreMesh(core_axis_name="c", subcore_axis_name="s")
    kfn = pl.kernel(
        _body,
        out_type=jax.ShapeDtypeStruct((NPAIR,), jnp.float32),
        mesh=mesh,
        compiler_params=pltpu.CompilerParams(needs_layout_passes=False),
        scratch_types=[
            pltpu.VMEM((SUP,), jnp.int32),
            pltpu.VMEM((SUP,), jnp.int32),
            pltpu.VMEM((SUP,), jnp.int32),
            pltpu.VMEM((SUP,), jnp.float32),
            pltpu.VMEM((SUP,), jnp.float32),
            pltpu.VMEM((3 * LANES, 17), jnp.float32),
            [pltpu.VMEM((2, CHUNK, DIM), jnp.int32) for _ in range(NBUF)],
            [pltpu.VMEM((CHUNK, DIM), jnp.int32) for _ in range(NBUF)],
            [pltpu.VMEM((2, CHUNK), jnp.float32) for _ in range(NBUF)],
            pltpu.SemaphoreType.DMA((NBUF,)),
        ],
    )
    return kfn(u, v, r, t, q, bs, bo, EV)


def _pack_bf16(M):
    n, d = M.shape
    return lax.bitcast_convert_type(
        M.astype(jnp.bfloat16).reshape(n, d // 2, 2), jnp.int32)


def kernel(u_idx, r_idx, v_idx, t, P, bs, bo, E_init, V_time, p_euc):
    u = jnp.asarray(u_idx, jnp.int32).reshape(NPAIR)
    v = jnp.asarray(v_idx, jnp.int32).reshape(NPAIR)
    r = jnp.asarray(r_idx, jnp.int32).reshape(NPAIR)
    tf = jnp.asarray(t, jnp.float32).reshape(NPAIR)
    q = _pack_bf16(jnp.concatenate([P, p_euc], axis=1))
    ev = _pack_bf16(jnp.concatenate([E_init, V_time], axis=1))
    out = _run(u, v, r, tf, q, bs, bo, ev)
    return out.reshape(B, L)


# speedup vs baseline: 2.4684x; 2.4684x over previous
import jax
import jax.numpy as jnp
from jax import lax
from jax.experimental import pallas as pl
from jax.experimental.pallas import tpu as pltpu
from jax.experimental.pallas import tpu_sc as plsc

NE = 100000
NR = 500
DIM = 128
B = 4096
L = 50

NC = 2
NS = 16
LANES = 16
NW = NC * NS

NPAIR = B * L
PER_W = NPAIR // NW
CHUNK = 32
SUP = 1600
NSUP = PER_W // SUP
CPS = SUP // CHUNK
NBUF = 2


def _body(u_hbm, v_hbm, r_hbm, t_hbm, q_hbm, bs_hbm, bo_hbm,
          E_hbm, V_hbm, out_hbm,
          uix, vix, rix, tv, outv, accbuf, rows, qr, bias, sems):
    wid = lax.axis_index("s") * NC + lax.axis_index("c")
    base = wid * PER_W
    lanes = lax.iota(jnp.int32, LANES)

    def copies(g, b):
        csl = pl.ds(g * CHUNK, CHUNK)
        Eu, Vu, Ev, Vv = (rows[b].at[i] for i in range(4))
        return [
            pltpu.make_async_copy(E_hbm.at[uix.at[csl]], Eu, sems.at[b]),
            pltpu.make_async_copy(V_hbm.at[uix.at[csl]], Vu, sems.at[b]),
            pltpu.make_async_copy(E_hbm.at[vix.at[csl]], Ev, sems.at[b]),
            pltpu.make_async_copy(V_hbm.at[vix.at[csl]], Vv, sems.at[b]),
            pltpu.make_async_copy(q_hbm.at[rix.at[csl]], qr[b], sems.at[b]),
            pltpu.make_async_copy(bs_hbm.at[uix.at[csl]], bias[b].at[0], sems.at[b]),
            pltpu.make_async_copy(bo_hbm.at[vix.at[csl]], bias[b].at[1], sems.at[b]),
        ]

    def fire(g, b):
        for cp in copies(g, b):
            cp.start()

    def drain(g, b):
        for cp in copies(g, b):
            cp.wait()

    def compute(g, b):
        Eu, Vu, Ev, Vv = (rows[b].at[i] for i in range(4))

        def group_body(gi, carry2):
            gsl = pl.ds(gi * LANES, LANES)

            @plsc.parallel_loop(0, LANES, unroll=2)
            def pair_iter(k):
                i = gi * LANES + k
                aa = jnp.zeros((LANES,), jnp.float32)
                ab = jnp.zeros((LANES,), jnp.float32)
                bb = jnp.zeros((LANES,), jnp.float32)
                for j4 in range(DIM // 32):
                    pz = plsc.bitcast(qr[b][i, pl.ds(16 * j4, 16)],
                                      jnp.bfloat16)
                    qz = plsc.bitcast(qr[b][i, pl.ds(DIM // 2 + 16 * j4, 16)],
                                      jnp.bfloat16)
                    pgs = plsc.unpack(pz, format=plsc.PackFormat.INTERLEAVED)
                    pps = plsc.unpack(qz, format=plsc.PackFormat.INTERLEAVED)
                    for h in range(2):
                        cs = pl.ds(32 * j4 + 16 * h, 16)
                        pg = pgs[h]
                        a = Eu[i, cs] * pg - Ev[i, cs] - pps[h]
                        bq = Vu[i, cs] * pg - Vv[i, cs]
                        aa = aa + a * a
                        ab = ab + a * bq
                        bb = bb + bq * bq
                accbuf[k, pl.ds(0, LANES)] = aa
                accbuf[k + LANES, pl.ds(0, LANES)] = ab
                accbuf[k + 2 * LANES, pl.ds(0, LANES)] = bb

            ta = jnp.zeros((LANES,), jnp.float32)
            tc = jnp.zeros((LANES,), jnp.float32)
            tb = jnp.zeros((LANES,), jnp.float32)
            for c in range(LANES):
                cc = jnp.full((LANES,), c, jnp.int32)
                ta = ta + plsc.load_gather(accbuf, [lanes, cc])
                tc = tc + plsc.load_gather(accbuf, [lanes + LANES, cc])
                tb = tb + plsc.load_gather(accbuf, [lanes + 2 * LANES, cc])
            osl = pl.ds(g * CHUNK + gi * LANES, LANES)
            tg = tv[osl]
            totals = ta + (tc + tc) * tg + tb * tg * tg
            outv[osl] = bias[b][0, gsl] + bias[b][1, gsl] - totals
            return carry2

        lax.fori_loop(0, CHUNK // LANES, group_body, 0, unroll=False)

    def super_body(s, carry):
        ssl = pl.ds(base + s * SUP, SUP)
        pltpu.sync_copy(u_hbm.at[ssl], uix)
        pltpu.sync_copy(v_hbm.at[ssl], vix)
        pltpu.sync_copy(r_hbm.at[ssl], rix)
        pltpu.sync_copy(t_hbm.at[ssl], tv)

        for b in range(NBUF):
            fire(b, b)

        def duo_body(g, carry2):
            for b in range(NBUF):
                gg = g + b
                drain(gg, b)
                compute(gg, b)

                @pl.when(gg + NBUF < CPS)
                def _():
                    fire(gg + NBUF, b)
            return carry2

        lax.fori_loop(0, CPS // NBUF, lambda i, c: duo_body(i * NBUF, c), 0,
                      unroll=False)
        pltpu.sync_copy(outv, out_hbm.at[ssl])
        return carry

    lax.fori_loop(0, NSUP, super_body, 0, unroll=False)


@jax.jit
def _run(u, v, r, t, q, bs, bo, E, V):
    mesh = plsc.VectorSubcoreMesh(core_axis_name="c", subcore_axis_name="s")
    kfn = pl.kernel(
        _body,
        out_type=jax.ShapeDtypeStruct((NPAIR,), jnp.float32),
        mesh=mesh,
        compiler_params=pltpu.CompilerParams(needs_layout_passes=False),
        scratch_types=[
            pltpu.VMEM((SUP,), jnp.int32),
            pltpu.VMEM((SUP,), jnp.int32),
            pltpu.VMEM((SUP,), jnp.int32),
            pltpu.VMEM((SUP,), jnp.float32),
            pltpu.VMEM((SUP,), jnp.float32),
            pltpu.VMEM((3 * LANES, 17), jnp.float32),
            [pltpu.VMEM((4, CHUNK, DIM), jnp.float32) for _ in range(NBUF)],
            [pltpu.VMEM((CHUNK, DIM), jnp.int32) for _ in range(NBUF)],
            [pltpu.VMEM((2, CHUNK), jnp.float32) for _ in range(NBUF)],
            pltpu.SemaphoreType.DMA((NBUF,)),
        ],
    )
    return kfn(u, v, r, t, q, bs, bo, E, V)


def _shuffle(M):
    return M.reshape(NR, DIM // 32, 2, 16).transpose(0, 1, 3, 2).reshape(NR, DIM)


def kernel(u_idx, r_idx, v_idx, t, P, bs, bo, E_init, V_time, p_euc):
    u = jnp.asarray(u_idx, jnp.int32).reshape(NPAIR)
    v = jnp.asarray(v_idx, jnp.int32).reshape(NPAIR)
    r = jnp.asarray(r_idx, jnp.int32).reshape(NPAIR)
    tf = jnp.asarray(t, jnp.float32).reshape(NPAIR)
    q = jnp.concatenate([_shuffle(P), _shuffle(p_euc)], axis=1)
    q = q.astype(jnp.bfloat16)
    q = lax.bitcast_convert_type(q.reshape(NR, DIM, 2), jnp.int32)
    out = _run(u, v, r, tf, q, bs, bo, E_init, V_time)
    return out.reshape(B, L)
